# Initial kernel scaffold; baseline (speedup 1.0000x reference)
import jax
import jax.numpy as jnp
from jax.experimental import pallas as pl


def kernel(vocab_ids, table):
    raise NotImplementedError("write your pallas kernel here")


# speedup vs baseline: 1.1135x; 1.1135x over previous
import functools

import jax
import jax.numpy as jnp
from jax import lax
from jax.experimental import pallas as pl
from jax.experimental.pallas import tpu as pltpu
from jax.experimental.pallas import tpu_sc as plsc

B = 16384
L = 50
D = 32
N = B * L

_info = plsc.get_sparse_core_info()
NC, NS = _info.num_cores, _info.num_subcores
NW = NC * NS
PER_W = N // NW
CH = 128
N_CH = PER_W // CH
SUP = 10
ROWS_SUP = SUP * CH
N_SUP = N_CH // SUP

_mesh = plsc.VectorSubcoreMesh(core_axis_name="c", subcore_axis_name="s")


@functools.partial(
    pl.kernel,
    mesh=_mesh,
    compiler_params=pltpu.CompilerParams(use_tc_tiling_on_sc=False),
    out_type=jax.ShapeDtypeStruct((N, D), jnp.float32),
    scratch_types=[
        pltpu.VMEM((N_CH, CH), jnp.int32),
        pltpu.VMEM((ROWS_SUP, D), jnp.float32),
        pltpu.VMEM((ROWS_SUP, D), jnp.float32),
        pltpu.SemaphoreType.DMA,
        pltpu.SemaphoreType.DMA,
        pltpu.SemaphoreType.DMA,
        pltpu.SemaphoreType.DMA,
    ],
)
def _sc_gather(idx_hbm, table_hbm, out_hbm, idx_v, buf_x, buf_y,
               gsem_x, gsem_y, ssem_x, ssem_y):
    wid = lax.axis_index("s") * NC + lax.axis_index("c")
    base = wid * PER_W

    pltpu.sync_copy(idx_hbm.at[pl.ds(wid * N_CH, N_CH)], idx_v)

    def gathers(sg, buf, sem):
        return [
            pltpu.make_async_copy(
                table_hbm.at[idx_v.at[sg * SUP + c]],
                buf.at[pl.ds(c * CH, CH)],
                sem,
            )
            for c in range(SUP)
        ]

    def scatter(sg, buf, sem):
        return pltpu.make_async_copy(
            buf, out_hbm.at[pl.ds(base + sg * ROWS_SUP, ROWS_SUP)], sem)

    for h in gathers(0, buf_x, gsem_x):
        h.start()

    def body(i, carry):
        s0 = 2 * i
        s1 = 2 * i + 1

        @pl.when(i > 0)
        def _():
            scatter(s1 - 2, buf_y, ssem_y).wait()

        for h in gathers(s1, buf_y, gsem_y):
            h.start()

        for h in gathers(s0, buf_x, gsem_x):
            h.wait()
        scatter(s0, buf_x, ssem_x).start()

        @pl.when(i < N_SUP // 2 - 1)
        def _():
            scatter(s0, buf_x, ssem_x).wait()
            for h in gathers(s0 + 2, buf_x, gsem_x):
                h.start()

        for h in gathers(s1, buf_y, gsem_y):
            h.wait()
        scatter(s1, buf_y, ssem_y).start()
        return carry

    lax.fori_loop(0, N_SUP // 2, body, 0)

    scatter(N_SUP - 2, buf_x, ssem_x).wait()
    scatter(N_SUP - 1, buf_y, ssem_y).wait()


def kernel(vocab_ids, table):
    idx = vocab_ids.reshape(N // CH, CH)
    out = _sc_gather(idx, table)
    return out.reshape(B, L, D)


# speedup vs baseline: 2.4129x; 2.1671x over previous
import functools

import jax
import jax.numpy as jnp
from jax import lax
from jax.experimental import pallas as pl
from jax.experimental.pallas import tpu as pltpu
from jax.experimental.pallas import tpu_sc as plsc

B = 16384
L = 50
D = 32
N = B * L

_info = plsc.get_sparse_core_info()
NC, NS = _info.num_cores, _info.num_subcores
NW = NC * NS
PER_W = N // NW
CH = 128
N_CH = PER_W // CH
SUP = 5
ROWS_SUP = SUP * CH
N_SUP = N_CH // SUP
N_PAIR = N_SUP // 2

_mesh = plsc.VectorSubcoreMesh(core_axis_name="c", subcore_axis_name="s")


@functools.partial(
    pl.kernel,
    mesh=_mesh,
    compiler_params=pltpu.CompilerParams(
        use_tc_tiling_on_sc=False, needs_layout_passes=False),
    out_type=jax.ShapeDtypeStruct((L, D, B), jnp.float32),
    scratch_types=[
        pltpu.VMEM((N_CH, CH), jnp.int32),
        pltpu.VMEM((ROWS_SUP, D), jnp.float32),
        pltpu.VMEM((ROWS_SUP, D), jnp.float32),
        pltpu.VMEM((SUP, D, CH), jnp.float32),
        pltpu.VMEM((SUP, D, CH), jnp.float32),
        pltpu.SemaphoreType.DMA,
        pltpu.SemaphoreType.DMA,
        pltpu.SemaphoreType.DMA,
        pltpu.SemaphoreType.DMA,
    ],
)
def _sc_gather(idx_hbm, table_hbm, out_hbm, idx_v, g_x, g_y, t_x, t_y,
               gsem_x, gsem_y, ssem_x, ssem_y):
    wid = lax.axis_index("s") * NC + lax.axis_index("c")

    pltpu.sync_copy(idx_hbm.at[pl.ds(wid * N_CH, N_CH)], idx_v)

    iota = lax.iota(jnp.int32, 16)

    def gathers(sg, buf, sem):
        return [
            pltpu.make_async_copy(
                table_hbm.at[idx_v.at[sg * SUP + c]],
                buf.at[pl.ds(c * CH, CH)],
                sem,
            )
            for c in range(SUP)
        ]

    def scatters(sg, tbuf, sem):
        hs = []
        for c in range(SUP):
            k = wid * N_CH + sg * SUP + c
            row_l = k // CH
            col = (k % CH) * CH
            hs.append(pltpu.make_async_copy(
                tbuf.at[c], out_hbm.at[row_l, :, pl.ds(col, CH)], sem))
        return hs

    def transpose(gbuf, tbuf):
        def one_chunk(c, carry):
            for g in range(CH // 16):
                rows = c * CH + g * 16 + iota
                for d in range(D):
                    cols = jnp.full((16,), d, jnp.int32)
                    tbuf[c, d, pl.ds(g * 16, 16)] = plsc.load_gather(
                        gbuf, [rows, cols])
            return carry
        lax.fori_loop(0, SUP, one_chunk, 0)

    for h in gathers(0, g_x, gsem_x):
        h.start()

    def body(i, carry):
        s0 = 2 * i
        s1 = 2 * i + 1

        @pl.when(i > 0)
        def _():
            for h in scatters(s1 - 2, t_y, ssem_y):
                h.wait()

        for h in gathers(s1, g_y, gsem_y):
            h.start()

        for h in gathers(s0, g_x, gsem_x):
            h.wait()

        @pl.when(i > 0)
        def _():
            for h in scatters(s0 - 2, t_x, ssem_x):
                h.wait()

        transpose(g_x, t_x)
        for h in scatters(s0, t_x, ssem_x):
            h.start()

        @pl.when(i < N_PAIR - 1)
        def _():
            for h in gathers(s0 + 2, g_x, gsem_x):
                h.start()

        for h in gathers(s1, g_y, gsem_y):
            h.wait()
        transpose(g_y, t_y)
        for h in scatters(s1, t_y, ssem_y):
            h.start()
        return carry

    lax.fori_loop(0, N_PAIR, body, 0)

    for h in scatters(N_SUP - 2, t_x, ssem_x):
        h.wait()
    for h in scatters(N_SUP - 1, t_y, ssem_y):
        h.wait()


def kernel(vocab_ids, table):
    idx = jnp.transpose(vocab_ids).reshape(N // CH, CH)
    out_t = _sc_gather(idx, table)
    return out_t.transpose(2, 0, 1)
